# Initial kernel scaffold; baseline (speedup 1.0000x reference)
import jax
import jax.numpy as jnp
from jax.experimental import pallas as pl


def kernel(x, W1, b1, W2, b2, codebook):
    raise NotImplementedError("write your pallas kernel here")


# speedup vs baseline: 1.2912x; 1.2912x over previous
import functools

import jax
import jax.numpy as jnp
from jax import lax
from jax.experimental import pallas as pl
from jax.experimental.pallas import tpu as pltpu
from jax.experimental.pallas import tpu_sc as plsc


_SPAN = 4096


def _bf16(a):
    return a.astype(jnp.bfloat16)


def _bf16_rt(a):
    return a.astype(jnp.bfloat16).astype(jnp.float32)


def _encode_block(x_ref, w1_ref, b1_ref, w2_ref, b2_ref, cbt_ref, zid_ref):
    h = jnp.tanh(
        jnp.dot(_bf16(x_ref[...]), _bf16(w1_ref[...]),
                preferred_element_type=jnp.float32)
        + b1_ref[...]
    )
    z = (
        jnp.dot(_bf16(h), _bf16(w2_ref[...]),
                preferred_element_type=jnp.float32)
        + b2_ref[...]
    )
    cbt = cbt_ref[...]
    cbsq = jnp.sum(cbt * cbt, axis=0)
    zsq = jnp.sum(z * z, axis=1, keepdims=True)
    mm = jnp.dot(_bf16(z), _bf16(cbt), preferred_element_type=jnp.float32)
    dist = zsq - 2.0 * mm + cbsq[None, :]

    k = dist.shape[1]
    acc = None
    idx = None
    for c in range(k // _SPAN):
        dc = dist[:, c * _SPAN:(c + 1) * _SPAN]
        m = jnp.min(dc, axis=1, keepdims=True)
        iota = lax.broadcasted_iota(jnp.int32, dc.shape, 1)
        im = jnp.min(jnp.where(dc <= m, iota, jnp.int32(_SPAN)), axis=1)
        gi = im + jnp.int32(c * _SPAN)
        mv = m[:, 0]
        if acc is None:
            acc, idx = _bf16_rt(mv), gi
        else:
            take = mv < acc
            acc = jnp.where(take, _bf16_rt(mv), acc)
            idx = jnp.where(take, gi, idx)
    zid_ref[...] = idx


def _encode_argmin(xf, W1, b1, W2, b2, cbT, block_rows):
    n, d_in = xf.shape
    d_h, k = cbT.shape
    grid = (n // block_rows,)
    return pl.pallas_call(
        _encode_block,
        grid=grid,
        in_specs=[
            pl.BlockSpec((block_rows, d_in), lambda i: (i, 0)),
            pl.BlockSpec((d_in, d_h), lambda i: (0, 0)),
            pl.BlockSpec((1, d_h), lambda i: (0, 0)),
            pl.BlockSpec((d_h, d_h), lambda i: (0, 0)),
            pl.BlockSpec((1, d_h), lambda i: (0, 0)),
            pl.BlockSpec((d_h, k), lambda i: (0, 0)),
        ],
        out_specs=pl.BlockSpec((block_rows,), lambda i: (i,)),
        out_shape=jax.ShapeDtypeStruct((n,), jnp.int32),
    )(xf, W1, b1.reshape(1, d_h), W2, b2.reshape(1, d_h), cbT)


_SC_CHUNK = 128
_SC_ROW = 128


@functools.cache
def _make_sc_gather(n_rows):
    d_h = _SC_ROW
    info = plsc.get_sparse_core_info()
    nw = info.num_cores * info.num_subcores
    rows_per_w = n_rows // nw
    chunks = rows_per_w // _SC_CHUNK
    mesh = plsc.VectorSubcoreMesh(core_axis_name="c", subcore_axis_name="s")

    @functools.partial(
        pl.kernel,
        mesh=mesh,
        out_type=jax.ShapeDtypeStruct((n_rows, d_h), jnp.float32),
        scratch_types=[
            pltpu.VMEM((_SC_CHUNK,), jnp.int32),
            pltpu.VMEM((_SC_CHUNK, d_h), jnp.float32),
            pltpu.SemaphoreType.DMA,
        ],
    )
    def gather(table_hbm, idx_hbm, out_hbm, idx_v, rows_v, sem):
        wid = lax.axis_index("s") * info.num_cores + lax.axis_index("c")
        for c in range(chunks):
            row = wid * chunks + c
            pltpu.sync_copy(idx_hbm.at[row], idx_v)
            pltpu.async_copy(table_hbm.at[idx_v], rows_v, sem).wait()
            pltpu.sync_copy(
                rows_v, out_hbm.at[pl.ds(row * _SC_CHUNK, _SC_CHUNK)]
            )

    return gather


def kernel(x, W1, b1, W2, b2, codebook):
    b, t, d_in = x.shape
    k, d_h = codebook.shape
    n = b * t
    xf = x.reshape(n, d_in)
    cbT = codebook.T
    zid = _encode_argmin(xf, W1, b1, W2, b2, cbT, block_rows=512)
    idx2d = zid.reshape(n // _SC_CHUNK, _SC_CHUNK)
    cb_pad = jnp.pad(codebook, ((0, 0), (0, _SC_ROW - d_h)))
    zq = _make_sc_gather(n)(cb_pad, idx2d)
    return zq[:, :d_h].reshape(b, t, d_h), zid.reshape(b, t)
